# Initial kernel scaffold; baseline (speedup 1.0000x reference)
import jax
import jax.numpy as jnp
from jax.experimental import pallas as pl


def kernel(obs, W_enc, b_enc, W_pre, b_pre, codebook, W_post, b_post, W_dec, b_dec):
    raise NotImplementedError("write your pallas kernel here")


# speedup vs baseline: 1.2190x; 1.2190x over previous
import functools

import jax
import jax.numpy as jnp
from jax import lax
from jax.experimental import pallas as pl
from jax.experimental.pallas import tpu as pltpu
from jax.experimental.pallas import tpu_sc as plsc

_B, _N, _OBS = 256, 64, 512
_T = _B * _N
_H = 1024
_E = 256
_V = 8192

_M = 256
_VC = 2048

_NC, _NS = 2, 16
_NW = _NC * _NS
_BPW = _T // _NW
_GCH = 128


def _encdist_body(obs_ref, we_ref, be_ref, wp_ref, bp_ref, cbt_ref,
                  z_ref, tok_ref):
    h = jax.nn.gelu(
        jnp.dot(obs_ref[...], we_ref[...], preferred_element_type=jnp.float32)
        + be_ref[...])
    z = jnp.dot(h.astype(jnp.bfloat16), wp_ref[...],
                preferred_element_type=jnp.float32) + bp_ref[...]
    z_ref[...] = z

    z_sq = jnp.sum(z * z, axis=1, keepdims=True)
    zb2 = (z + z).astype(jnp.bfloat16)

    run_min = jnp.full((_M, 1), jnp.inf, jnp.float32)
    run_idx = jnp.zeros((_M, 1), jnp.float32)
    for c in range(_V // _VC):
        cb_c = cbt_ref[:, c * _VC:(c + 1) * _VC]
        mm2 = jnp.dot(zb2, cb_c, preferred_element_type=jnp.float32)
        cc = cb_c.astype(jnp.float32)
        c_sq = jnp.sum(cc * cc, axis=0, keepdims=True)
        dist = (z_sq + c_sq) - mm2
        m = jnp.min(dist, axis=1, keepdims=True)
        iota = (lax.broadcasted_iota(jnp.int32, (1, _VC), 1)
                + (c * _VC)).astype(jnp.float32)
        cand = jnp.where(dist == m, iota, jnp.float32(1e9))
        idx_c = jnp.min(cand, axis=1, keepdims=True)
        better = m < run_min
        run_idx = jnp.where(better, idx_c, run_idx)
        run_min = jnp.where(better, m, run_min)
    tok_ref[...] = run_idx.astype(jnp.int32)


def _dec_body(z_ref, zq_ref, wpost_ref, bpost_ref, wdec_ref, bdec_ref,
              rec_ref):
    di = z_ref[...] + (zq_ref[...] - z_ref[...])
    d = jax.nn.gelu(
        jnp.dot(di.astype(jnp.bfloat16), wpost_ref[...],
                preferred_element_type=jnp.float32) + bpost_ref[...])
    rec_ref[...] = jnp.dot(d.astype(jnp.bfloat16), wdec_ref[...],
                           preferred_element_type=jnp.float32) + bdec_ref[...]


def _encdist(obs_bf, we_bf, be, wp_bf, bp, cbt_bf):
    grid = (_T // _M,)
    return pl.pallas_call(
        _encdist_body,
        grid=grid,
        in_specs=[
            pl.BlockSpec((_M, _OBS), lambda i: (i, 0)),
            pl.BlockSpec((_OBS, _H), lambda i: (0, 0)),
            pl.BlockSpec((1, _H), lambda i: (0, 0)),
            pl.BlockSpec((_H, _E), lambda i: (0, 0)),
            pl.BlockSpec((1, _E), lambda i: (0, 0)),
            pl.BlockSpec((_E, _V), lambda i: (0, 0)),
        ],
        out_specs=[
            pl.BlockSpec((_M, _E), lambda i: (i, 0)),
            pl.BlockSpec((_M, 1), lambda i: (i, 0)),
        ],
        out_shape=[
            jax.ShapeDtypeStruct((_T, _E), jnp.float32),
            jax.ShapeDtypeStruct((_T, 1), jnp.int32),
        ],
        compiler_params=pltpu.CompilerParams(
            dimension_semantics=("arbitrary",)),
    )(obs_bf, we_bf, be, wp_bf, bp, cbt_bf)


def _decode(z, zq, wpost_bf, bpost, wdec_bf, bdec):
    grid = (_T // _M,)
    return pl.pallas_call(
        _dec_body,
        grid=grid,
        in_specs=[
            pl.BlockSpec((_M, _E), lambda i: (i, 0)),
            pl.BlockSpec((_M, _E), lambda i: (i, 0)),
            pl.BlockSpec((_E, _H), lambda i: (0, 0)),
            pl.BlockSpec((1, _H), lambda i: (0, 0)),
            pl.BlockSpec((_H, _OBS), lambda i: (0, 0)),
            pl.BlockSpec((1, _OBS), lambda i: (0, 0)),
        ],
        out_specs=pl.BlockSpec((_M, _OBS), lambda i: (i, 0)),
        out_shape=jax.ShapeDtypeStruct((_T, _OBS), jnp.float32),
        compiler_params=pltpu.CompilerParams(
            dimension_semantics=("arbitrary",)),
    )(z, zq, wpost_bf, bpost, wdec_bf, bdec)


def _gather(codebook, tokens):
    mesh = plsc.VectorSubcoreMesh(core_axis_name="c", subcore_axis_name="s")

    @functools.partial(
        pl.kernel, mesh=mesh,
        out_type=jax.ShapeDtypeStruct((_T, _E), jnp.float32),
        scratch_types=[
            pltpu.VMEM((_GCH,), jnp.int32),
            pltpu.VMEM((_GCH, _E), jnp.float32),
            pltpu.SemaphoreType.DMA,
        ],
    )
    def k(table_hbm, idx_hbm, out_hbm, idx_v, rows_v, sem):
        wid = lax.axis_index("s") * _NC + lax.axis_index("c")
        base = wid * _BPW
        for c in range(_BPW // _GCH):
            off = base + c * _GCH
            pltpu.sync_copy(idx_hbm.at[pl.ds(off, _GCH)], idx_v)
            pltpu.async_copy(table_hbm.at[idx_v], rows_v, sem).wait()
            pltpu.sync_copy(rows_v, out_hbm.at[pl.ds(off, _GCH)])

    return k(codebook, tokens)


def kernel(obs, W_enc, b_enc, W_pre, b_pre, codebook, W_post, b_post,
           W_dec, b_dec):
    obs_bf = obs.reshape(_T, _OBS).astype(jnp.bfloat16)
    we_bf = W_enc.astype(jnp.bfloat16)
    wp_bf = W_pre.astype(jnp.bfloat16)
    cbt_bf = codebook.T.astype(jnp.bfloat16)
    wpost_bf = W_post.astype(jnp.bfloat16)
    wdec_bf = W_dec.astype(jnp.bfloat16)

    z_flat, tok2d = _encdist(obs_bf, we_bf, b_enc.reshape(1, _H),
                             wp_bf, b_pre.reshape(1, _E), cbt_bf)
    tokens = tok2d.reshape(_T)
    zq_flat = _gather(codebook, tokens)
    rec_flat = _decode(z_flat, zq_flat, wpost_bf, b_post.reshape(1, _H),
                       wdec_bf, b_dec.reshape(1, _OBS))
    return (z_flat.reshape(_B, _N, _E),
            zq_flat.reshape(_B, _N, _E),
            rec_flat.reshape(_B, _N, _OBS))
